# baseline (speedup 1.0000x reference)
import functools

import jax
import jax.numpy as jnp
from jax import lax
from jax.experimental import pallas as pl
from jax.experimental.pallas import tpu as pltpu
from jax.experimental.pallas import tpu_sc as plsc

N = 10000
E = 320000
DIN = 128
DH = 64
DF = 128
DOUT = 6
G = 128

NC = 2
NS = 16
NW = NC * NS
CHUNK = 128
NBUF = 2
S0 = 32
S1 = 128
H0 = S0 // 2
H1 = S1 // 2
TCH = NS * (S0 + S1)
CH0 = NS * S0
EP = TCH * CHUNK
NPAD = 10112
RPW = NPAD // NS


def _sc_agg(h, src, dst, zeros):
    mesh = plsc.VectorSubcoreMesh(core_axis_name="c", subcore_axis_name="s")

    @functools.partial(
        pl.kernel,
        mesh=mesh,
        out_type=jax.ShapeDtypeStruct((NC * NPAD, DF), jnp.float32),
        scratch_types=[
            pltpu.VMEM((max(H0, H1), CHUNK), jnp.int32),
            pltpu.VMEM((max(H0, H1), CHUNK), jnp.int32),
            pltpu.VMEM((CHUNK, DF), jnp.float32),
            pltpu.VMEM((CHUNK, DF), jnp.float32),
            pltpu.VMEM_SHARED((NPAD, DF), jnp.float32),
            pltpu.SemaphoreType.DMA,
            pltpu.SemaphoreType.DMA,
            pltpu.SemaphoreType.DMA,
            pltpu.SemaphoreType.DMA,
            pltpu.SemaphoreType.DMA,
        ],
    )
    def k(h_hbm, src_hbm, dst_hbm, z_hbm, out_hbm, srcv, dstv,
          rows0, rows1, acc, isem, gs0, gs1, ss0, ss1):
        c = lax.axis_index("c")
        s = lax.axis_index("s")
        pltpu.sync_copy(z_hbm.at[pl.ds(s * RPW, RPW)], acc.at[pl.ds(s * RPW, RPW)])
        plsc.subcore_barrier()

        def run(base_chunk, hsteps):
            for half in range(2):
                b = base_chunk + half * hsteps
                ic0 = pltpu.async_copy(
                    src_hbm.at[pl.ds(b, hsteps)], srcv.at[pl.ds(0, hsteps)],
                    isem)
                ic1 = pltpu.async_copy(
                    dst_hbm.at[pl.ds(b, hsteps)], dstv.at[pl.ds(0, hsteps)],
                    isem)
                ic0.wait()
                ic1.wait()

                @pl.loop(0, hsteps, step=2)
                def _(t):
                    g0 = pltpu.async_copy(h_hbm.at[srcv.at[t]], rows0, gs0)
                    g1 = pltpu.async_copy(h_hbm.at[srcv.at[t + 1]], rows1, gs1)
                    g0.wait()
                    s0 = pltpu.async_copy(rows0, acc.at[dstv.at[t]], ss0,
                                          add=True)
                    g1.wait()
                    s1 = pltpu.async_copy(rows1, acc.at[dstv.at[t + 1]], ss1,
                                          add=True)
                    s0.wait()
                    s1.wait()

        @pl.when(c == 0)
        def _():
            run(s * S0, H0)

        @pl.when(c == 1)
        def _():
            run(CH0 + s * S1, H1)

        plsc.subcore_barrier()
        pltpu.sync_copy(
            acc.at[pl.ds(s * RPW, RPW)],
            out_hbm.at[pl.ds(c * NPAD + s * RPW, RPW)],
        )

    return k(h, src, dst, zeros)


_ROWS = 2000


def _tc_pre(x, W, b):
    def body(x_ref, w_ref, b_ref, o_ref):
        t = (
            jnp.dot(x_ref[...], w_ref[...], preferred_element_type=jnp.float32)
            + b_ref[...]
        )
        o_ref[...] = jnp.concatenate(
            [t, jnp.zeros((_ROWS, DF - DH), jnp.float32)], axis=1
        )

    return pl.pallas_call(
        body,
        grid=(N // _ROWS,),
        in_specs=[
            pl.BlockSpec((_ROWS, DIN), lambda i: (i, 0)),
            pl.BlockSpec((DIN, DH), lambda i: (0, 0)),
            pl.BlockSpec((1, DH), lambda i: (0, 0)),
        ],
        out_specs=pl.BlockSpec((_ROWS, DF), lambda i: (i, 0)),
        out_shape=jax.ShapeDtypeStruct((N, DF), jnp.float32),
    )(x, W, b.reshape(1, DH))


def _tc_mlp(h, a0, a1, Wa, ba, Wb, bb):
    def body(h_ref, a0_ref, a1_ref, wa, bar, wb, bbr, o_ref):
        t = h_ref[...] + a0_ref[...] + a1_ref[...]
        t = t[:, :DH]
        t = jnp.maximum(
            jnp.dot(t, wa[...], preferred_element_type=jnp.float32) + bar[...], 0.0
        )
        t = jnp.dot(t, wb[...], preferred_element_type=jnp.float32) + bbr[...]
        t = jnp.maximum(t, 0.0)
        o_ref[...] = jnp.concatenate(
            [t, jnp.zeros((_ROWS, DF - DH), jnp.float32)], axis=1
        )

    rows_spec = pl.BlockSpec((_ROWS, DF), lambda i: (i, 0))
    w_spec = pl.BlockSpec((DH, DH), lambda i: (0, 0))
    b_spec = pl.BlockSpec((1, DH), lambda i: (0, 0))
    return pl.pallas_call(
        body,
        grid=(N // _ROWS,),
        in_specs=[rows_spec, rows_spec, rows_spec, w_spec, b_spec, w_spec, b_spec],
        out_specs=rows_spec,
        out_shape=jax.ShapeDtypeStruct((N, DF), jnp.float32),
    )(h, a0, a1, Wa, ba.reshape(1, DH), Wb, bb.reshape(1, DH))


def _tc_pool_post(h, batch3, Wp1, bp1, Wp2, bp2):
    nb = N // _ROWS

    def body(h_ref, b_ref, w1, b1r, w2, b2r, o_ref, acc):
        i = pl.program_id(0)

        @pl.when(i == 0)
        def _():
            acc[...] = jnp.zeros_like(acc)

        ids = b_ref[0]
        gi = lax.broadcasted_iota(jnp.int32, (G, _ROWS), 0)
        onehot = (gi == ids).astype(jnp.float32)
        acc[...] += jnp.dot(
            onehot, h_ref[...][:, :DH], preferred_element_type=jnp.float32
        )

        @pl.when(i == nb - 1)
        def _():
            p = acc[...]
            t = jnp.maximum(
                jnp.dot(p, w1[...], preferred_element_type=jnp.float32) + b1r[...],
                0.0,
            )
            o = jnp.dot(t, w2[...], preferred_element_type=jnp.float32) + b2r[...]
            m = jnp.max(o, axis=1, keepdims=True)
            lse = jnp.log(jnp.sum(jnp.exp(o - m), axis=1, keepdims=True)) + m
            o_ref[...] = o - lse

    return pl.pallas_call(
        body,
        grid=(nb,),
        in_specs=[
            pl.BlockSpec((_ROWS, DF), lambda i: (i, 0)),
            pl.BlockSpec((1, 1, _ROWS), lambda i: (i, 0, 0)),
            pl.BlockSpec((DH, DH), lambda i: (0, 0)),
            pl.BlockSpec((1, DH), lambda i: (0, 0)),
            pl.BlockSpec((DH, DOUT), lambda i: (0, 0)),
            pl.BlockSpec((1, DOUT), lambda i: (0, 0)),
        ],
        out_specs=pl.BlockSpec((G, DOUT), lambda i: (0, 0)),
        out_shape=jax.ShapeDtypeStruct((G, DOUT), jnp.float32),
        scratch_shapes=[pltpu.VMEM((G, DH), jnp.float32)],
    )(h, batch3, Wp1, bp1.reshape(1, DH), Wp2, bp2.reshape(1, DOUT))


def kernel(x, edge_index, batch, W_pre, b_pre, W1a, b1a, W1b, b1b, W2a, b2a,
           W2b, b2b, W3a, b3a, W3b, b3b, Wp1, bp1, Wp2, bp2):
    pad = EP - E
    src = jnp.concatenate([edge_index[0], jnp.zeros((pad,), jnp.int32)])
    pad_dst = N + jnp.arange(pad, dtype=jnp.int32) % (NPAD - N)
    dst = jnp.concatenate([edge_index[1], pad_dst])
    src = src.reshape(TCH, CHUNK)
    dst = dst.reshape(TCH, CHUNK)
    zeros = jnp.zeros((NPAD, DF), jnp.float32)
    batch3 = batch.reshape(N // _ROWS, 1, _ROWS)

    h = _tc_pre(x, W_pre, b_pre)
    for Wa, ba, Wb, bb in ((W1a, b1a, W1b, b1b), (W2a, b2a, W2b, b2b),
                           (W3a, b3a, W3b, b3b)):
        parts = _sc_agg(h, src, dst, zeros)
        a0 = lax.slice(parts, (0, 0), (N, DF))
        a1 = lax.slice(parts, (NPAD, 0), (NPAD + N, DF))
        h = _tc_mlp(h, a0, a1, Wa, ba, Wb, bb)

    return _tc_pool_post(h, batch3, Wp1, bp1, Wp2, bp2)


# speedup vs baseline: 1.4279x; 1.4279x over previous
import functools

import jax
import jax.numpy as jnp
from jax import lax
from jax.experimental import pallas as pl
from jax.experimental.pallas import tpu as pltpu
from jax.experimental.pallas import tpu_sc as plsc

N = 10000
E = 320000
DIN = 128
DH = 64
DF = 128
DOUT = 6
G = 128

NC = 2
NS = 16
NW = NC * NS
CHUNK = 128
NBUF = 2
S0 = 144
S1 = 16
SEG0 = 48
SEG1 = 16
TCH = NS * (S0 + S1)
CH0 = NS * S0
EP = TCH * CHUNK
NPAD = 10112
RPW = NPAD // NS


def _sc_agg(h, src, dst, zeros):
    mesh = plsc.VectorSubcoreMesh(core_axis_name="c", subcore_axis_name="s")

    @functools.partial(
        pl.kernel,
        mesh=mesh,
        out_type=jax.ShapeDtypeStruct((NC * NPAD, DF), jnp.float32),
        scratch_types=[
            pltpu.VMEM((max(SEG0, SEG1), CHUNK), jnp.int32),
            pltpu.VMEM((max(SEG0, SEG1), CHUNK), jnp.int32),
            pltpu.VMEM((CHUNK, DF), jnp.float32),
            pltpu.VMEM((CHUNK, DF), jnp.float32),
            pltpu.VMEM_SHARED((NPAD, DF), jnp.float32),
            pltpu.SemaphoreType.DMA,
            pltpu.SemaphoreType.DMA,
            pltpu.SemaphoreType.DMA,
            pltpu.SemaphoreType.DMA,
            pltpu.SemaphoreType.DMA,
        ],
    )
    def k(h_hbm, src_hbm, dst_hbm, z_hbm, out_hbm, srcv, dstv,
          rows0, rows1, acc, isem, gs0, gs1, ss0, ss1):
        c = lax.axis_index("c")
        s = lax.axis_index("s")
        pltpu.sync_copy(z_hbm.at[pl.ds(s * RPW, RPW)], acc.at[pl.ds(s * RPW, RPW)])
        plsc.subcore_barrier()

        def run(base_chunk, nseg, hsteps):
            for seg in range(nseg):
                b = base_chunk + seg * hsteps
                ic0 = pltpu.async_copy(
                    src_hbm.at[pl.ds(b, hsteps)], srcv.at[pl.ds(0, hsteps)],
                    isem)
                ic1 = pltpu.async_copy(
                    dst_hbm.at[pl.ds(b, hsteps)], dstv.at[pl.ds(0, hsteps)],
                    isem)
                ic0.wait()
                ic1.wait()

                @pl.loop(0, hsteps, step=2)
                def _(t):
                    g0 = pltpu.async_copy(h_hbm.at[srcv.at[t]], rows0, gs0)
                    g1 = pltpu.async_copy(h_hbm.at[srcv.at[t + 1]], rows1, gs1)
                    g0.wait()
                    s0 = pltpu.async_copy(rows0, acc.at[dstv.at[t]], ss0,
                                          add=True)
                    g1.wait()
                    s1 = pltpu.async_copy(rows1, acc.at[dstv.at[t + 1]], ss1,
                                          add=True)
                    s0.wait()
                    s1.wait()

        @pl.when(c == 0)
        def _():
            run(s * S0, S0 // SEG0, SEG0)

        @pl.when(c == 1)
        def _():
            run(CH0 + s * S1, S1 // SEG1, SEG1)

        plsc.subcore_barrier()
        pltpu.sync_copy(
            acc.at[pl.ds(s * RPW, RPW)],
            out_hbm.at[pl.ds(c * NPAD + s * RPW, RPW)],
        )

    return k(h, src, dst, zeros)


_ROWS = 2000


def _tc_pre(x, W, b):
    def body(x_ref, w_ref, b_ref, o_ref):
        t = (
            jnp.dot(x_ref[...], w_ref[...], preferred_element_type=jnp.float32)
            + b_ref[...]
        )
        o_ref[...] = jnp.concatenate(
            [t, jnp.zeros((_ROWS, DF - DH), jnp.float32)], axis=1
        )

    return pl.pallas_call(
        body,
        grid=(N // _ROWS,),
        in_specs=[
            pl.BlockSpec((_ROWS, DIN), lambda i: (i, 0)),
            pl.BlockSpec((DIN, DH), lambda i: (0, 0)),
            pl.BlockSpec((1, DH), lambda i: (0, 0)),
        ],
        out_specs=pl.BlockSpec((_ROWS, DF), lambda i: (i, 0)),
        out_shape=jax.ShapeDtypeStruct((N, DF), jnp.float32),
    )(x, W, b.reshape(1, DH))


def _tc_mlp(h, a0, a1, Wa, ba, Wb, bb):
    def body(h_ref, a0_ref, a1_ref, wa, bar, wb, bbr, o_ref):
        t = h_ref[...] + a0_ref[...] + a1_ref[...]
        t = t[:, :DH]
        t = jnp.maximum(
            jnp.dot(t, wa[...], preferred_element_type=jnp.float32) + bar[...], 0.0
        )
        t = jnp.dot(t, wb[...], preferred_element_type=jnp.float32) + bbr[...]
        t = jnp.maximum(t, 0.0)
        o_ref[...] = jnp.concatenate(
            [t, jnp.zeros((_ROWS, DF - DH), jnp.float32)], axis=1
        )

    rows_spec = pl.BlockSpec((_ROWS, DF), lambda i: (i, 0))
    w_spec = pl.BlockSpec((DH, DH), lambda i: (0, 0))
    b_spec = pl.BlockSpec((1, DH), lambda i: (0, 0))
    return pl.pallas_call(
        body,
        grid=(N // _ROWS,),
        in_specs=[rows_spec, rows_spec, rows_spec, w_spec, b_spec, w_spec, b_spec],
        out_specs=rows_spec,
        out_shape=jax.ShapeDtypeStruct((N, DF), jnp.float32),
    )(h, a0, a1, Wa, ba.reshape(1, DH), Wb, bb.reshape(1, DH))


def _tc_pool_post(h, batch3, Wp1, bp1, Wp2, bp2):
    nb = N // _ROWS

    def body(h_ref, b_ref, w1, b1r, w2, b2r, o_ref, acc):
        i = pl.program_id(0)

        @pl.when(i == 0)
        def _():
            acc[...] = jnp.zeros_like(acc)

        ids = b_ref[0]
        gi = lax.broadcasted_iota(jnp.int32, (G, _ROWS), 0)
        onehot = (gi == ids).astype(jnp.float32)
        acc[...] += jnp.dot(
            onehot, h_ref[...][:, :DH], preferred_element_type=jnp.float32
        )

        @pl.when(i == nb - 1)
        def _():
            p = acc[...]
            t = jnp.maximum(
                jnp.dot(p, w1[...], preferred_element_type=jnp.float32) + b1r[...],
                0.0,
            )
            o = jnp.dot(t, w2[...], preferred_element_type=jnp.float32) + b2r[...]
            m = jnp.max(o, axis=1, keepdims=True)
            lse = jnp.log(jnp.sum(jnp.exp(o - m), axis=1, keepdims=True)) + m
            o_ref[...] = o - lse

    return pl.pallas_call(
        body,
        grid=(nb,),
        in_specs=[
            pl.BlockSpec((_ROWS, DF), lambda i: (i, 0)),
            pl.BlockSpec((1, 1, _ROWS), lambda i: (i, 0, 0)),
            pl.BlockSpec((DH, DH), lambda i: (0, 0)),
            pl.BlockSpec((1, DH), lambda i: (0, 0)),
            pl.BlockSpec((DH, DOUT), lambda i: (0, 0)),
            pl.BlockSpec((1, DOUT), lambda i: (0, 0)),
        ],
        out_specs=pl.BlockSpec((G, DOUT), lambda i: (0, 0)),
        out_shape=jax.ShapeDtypeStruct((G, DOUT), jnp.float32),
        scratch_shapes=[pltpu.VMEM((G, DH), jnp.float32)],
    )(h, batch3, Wp1, bp1.reshape(1, DH), Wp2, bp2.reshape(1, DOUT))


def kernel(x, edge_index, batch, W_pre, b_pre, W1a, b1a, W1b, b1b, W2a, b2a,
           W2b, b2b, W3a, b3a, W3b, b3b, Wp1, bp1, Wp2, bp2):
    pad = EP - E
    src = jnp.concatenate([edge_index[0], jnp.zeros((pad,), jnp.int32)])
    pad_dst = N + jnp.arange(pad, dtype=jnp.int32) % (NPAD - N)
    dst = jnp.concatenate([edge_index[1], pad_dst])
    src = src.reshape(TCH, CHUNK)
    dst = dst.reshape(TCH, CHUNK)
    zeros = jnp.zeros((NPAD, DF), jnp.float32)
    batch3 = batch.reshape(N // _ROWS, 1, _ROWS)

    h = _tc_pre(x, W_pre, b_pre)
    for Wa, ba, Wb, bb in ((W1a, b1a, W1b, b1b), (W2a, b2a, W2b, b2b),
                           (W3a, b3a, W3b, b3b)):
        parts = _sc_agg(h, src, dst, zeros)
        a0 = lax.slice(parts, (0, 0), (N, DF))
        a1 = lax.slice(parts, (NPAD, 0), (NPAD + N, DF))
        h = _tc_mlp(h, a0, a1, Wa, ba, Wb, bb)

    return _tc_pool_post(h, batch3, Wp1, bp1, Wp2, bp2)


# speedup vs baseline: 1.4372x; 1.0065x over previous
import functools

import jax
import jax.numpy as jnp
from jax import lax
from jax.experimental import pallas as pl
from jax.experimental.pallas import tpu as pltpu
from jax.experimental.pallas import tpu_sc as plsc

N = 10000
E = 320000
DIN = 128
DH = 64
DF = 128
DOUT = 6
G = 128

NC = 2
NS = 16
NW = NC * NS
CHUNK = 128
NBUF = 2
S0 = 152
S1 = 8
SEGS0 = (48, 48, 48, 8)
SEGS1 = (8,)
SEGMAX = 48
TCH = NS * (S0 + S1)
CH0 = NS * S0
EP = TCH * CHUNK
NPAD = 10112
RPW = NPAD // NS


def _sc_agg(h, src, dst, zeros):
    mesh = plsc.VectorSubcoreMesh(core_axis_name="c", subcore_axis_name="s")

    @functools.partial(
        pl.kernel,
        mesh=mesh,
        out_type=jax.ShapeDtypeStruct((NC * NPAD, DF), jnp.float32),
        scratch_types=[
            pltpu.VMEM((SEGMAX, CHUNK), jnp.int32),
            pltpu.VMEM((SEGMAX, CHUNK), jnp.int32),
            pltpu.VMEM((CHUNK, DF), jnp.float32),
            pltpu.VMEM((CHUNK, DF), jnp.float32),
            pltpu.VMEM_SHARED((NPAD, DF), jnp.float32),
            pltpu.SemaphoreType.DMA,
            pltpu.SemaphoreType.DMA,
            pltpu.SemaphoreType.DMA,
            pltpu.SemaphoreType.DMA,
            pltpu.SemaphoreType.DMA,
        ],
    )
    def k(h_hbm, src_hbm, dst_hbm, z_hbm, out_hbm, srcv, dstv,
          rows0, rows1, acc, isem, gs0, gs1, ss0, ss1):
        c = lax.axis_index("c")
        s = lax.axis_index("s")
        pltpu.sync_copy(z_hbm.at[pl.ds(s * RPW, RPW)], acc.at[pl.ds(s * RPW, RPW)])
        plsc.subcore_barrier()

        def run(base_chunk, segs):
            off = 0
            for hsteps in segs:
                b = base_chunk + off
                off += hsteps
                ic0 = pltpu.async_copy(
                    src_hbm.at[pl.ds(b, hsteps)], srcv.at[pl.ds(0, hsteps)],
                    isem)
                ic1 = pltpu.async_copy(
                    dst_hbm.at[pl.ds(b, hsteps)], dstv.at[pl.ds(0, hsteps)],
                    isem)
                ic0.wait()
                ic1.wait()

                @pl.loop(0, hsteps, step=2)
                def _(t):
                    g0 = pltpu.async_copy(h_hbm.at[srcv.at[t]], rows0, gs0)
                    g1 = pltpu.async_copy(h_hbm.at[srcv.at[t + 1]], rows1, gs1)
                    g0.wait()
                    s0 = pltpu.async_copy(rows0, acc.at[dstv.at[t]], ss0,
                                          add=True)
                    g1.wait()
                    s1 = pltpu.async_copy(rows1, acc.at[dstv.at[t + 1]], ss1,
                                          add=True)
                    s0.wait()
                    s1.wait()

        @pl.when(c == 0)
        def _():
            run(s * S0, SEGS0)

        @pl.when(c == 1)
        def _():
            run(CH0 + s * S1, SEGS1)

        plsc.subcore_barrier()
        pltpu.sync_copy(
            acc.at[pl.ds(s * RPW, RPW)],
            out_hbm.at[pl.ds(c * NPAD + s * RPW, RPW)],
        )

    return k(h, src, dst, zeros)


_ROWS = 2000


def _tc_pre(x, W, b):
    def body(x_ref, w_ref, b_ref, o_ref):
        t = (
            jnp.dot(x_ref[...], w_ref[...], preferred_element_type=jnp.float32)
            + b_ref[...]
        )
        o_ref[...] = jnp.concatenate(
            [t, jnp.zeros((_ROWS, DF - DH), jnp.float32)], axis=1
        )

    return pl.pallas_call(
        body,
        grid=(N // _ROWS,),
        in_specs=[
            pl.BlockSpec((_ROWS, DIN), lambda i: (i, 0)),
            pl.BlockSpec((DIN, DH), lambda i: (0, 0)),
            pl.BlockSpec((1, DH), lambda i: (0, 0)),
        ],
        out_specs=pl.BlockSpec((_ROWS, DF), lambda i: (i, 0)),
        out_shape=jax.ShapeDtypeStruct((N, DF), jnp.float32),
    )(x, W, b.reshape(1, DH))


def _tc_mlp(h, a0, a1, Wa, ba, Wb, bb):
    def body(h_ref, a0_ref, a1_ref, wa, bar, wb, bbr, o_ref):
        t = h_ref[...] + a0_ref[...] + a1_ref[...]
        t = t[:, :DH]
        t = jnp.maximum(
            jnp.dot(t, wa[...], preferred_element_type=jnp.float32) + bar[...], 0.0
        )
        t = jnp.dot(t, wb[...], preferred_element_type=jnp.float32) + bbr[...]
        t = jnp.maximum(t, 0.0)
        o_ref[...] = jnp.concatenate(
            [t, jnp.zeros((_ROWS, DF - DH), jnp.float32)], axis=1
        )

    rows_spec = pl.BlockSpec((_ROWS, DF), lambda i: (i, 0))
    w_spec = pl.BlockSpec((DH, DH), lambda i: (0, 0))
    b_spec = pl.BlockSpec((1, DH), lambda i: (0, 0))
    return pl.pallas_call(
        body,
        grid=(N // _ROWS,),
        in_specs=[rows_spec, rows_spec, rows_spec, w_spec, b_spec, w_spec, b_spec],
        out_specs=rows_spec,
        out_shape=jax.ShapeDtypeStruct((N, DF), jnp.float32),
    )(h, a0, a1, Wa, ba.reshape(1, DH), Wb, bb.reshape(1, DH))


def _tc_pool_post(h, batch3, Wp1, bp1, Wp2, bp2):
    nb = N // _ROWS

    def body(h_ref, b_ref, w1, b1r, w2, b2r, o_ref, acc):
        i = pl.program_id(0)

        @pl.when(i == 0)
        def _():
            acc[...] = jnp.zeros_like(acc)

        ids = b_ref[0]
        gi = lax.broadcasted_iota(jnp.int32, (G, _ROWS), 0)
        onehot = (gi == ids).astype(jnp.float32)
        acc[...] += jnp.dot(
            onehot, h_ref[...][:, :DH], preferred_element_type=jnp.float32
        )

        @pl.when(i == nb - 1)
        def _():
            p = acc[...]
            t = jnp.maximum(
                jnp.dot(p, w1[...], preferred_element_type=jnp.float32) + b1r[...],
                0.0,
            )
            o = jnp.dot(t, w2[...], preferred_element_type=jnp.float32) + b2r[...]
            m = jnp.max(o, axis=1, keepdims=True)
            lse = jnp.log(jnp.sum(jnp.exp(o - m), axis=1, keepdims=True)) + m
            o_ref[...] = o - lse

    return pl.pallas_call(
        body,
        grid=(nb,),
        in_specs=[
            pl.BlockSpec((_ROWS, DF), lambda i: (i, 0)),
            pl.BlockSpec((1, 1, _ROWS), lambda i: (i, 0, 0)),
            pl.BlockSpec((DH, DH), lambda i: (0, 0)),
            pl.BlockSpec((1, DH), lambda i: (0, 0)),
            pl.BlockSpec((DH, DOUT), lambda i: (0, 0)),
            pl.BlockSpec((1, DOUT), lambda i: (0, 0)),
        ],
        out_specs=pl.BlockSpec((G, DOUT), lambda i: (0, 0)),
        out_shape=jax.ShapeDtypeStruct((G, DOUT), jnp.float32),
        scratch_shapes=[pltpu.VMEM((G, DH), jnp.float32)],
    )(h, batch3, Wp1, bp1.reshape(1, DH), Wp2, bp2.reshape(1, DOUT))


def kernel(x, edge_index, batch, W_pre, b_pre, W1a, b1a, W1b, b1b, W2a, b2a,
           W2b, b2b, W3a, b3a, W3b, b3b, Wp1, bp1, Wp2, bp2):
    pad = EP - E
    src = jnp.concatenate([edge_index[0], jnp.zeros((pad,), jnp.int32)])
    pad_dst = N + jnp.arange(pad, dtype=jnp.int32) % (NPAD - N)
    dst = jnp.concatenate([edge_index[1], pad_dst])
    src = src.reshape(TCH, CHUNK)
    dst = dst.reshape(TCH, CHUNK)
    zeros = jnp.zeros((NPAD, DF), jnp.float32)
    batch3 = batch.reshape(N // _ROWS, 1, _ROWS)

    h = _tc_pre(x, W_pre, b_pre)
    for Wa, ba, Wb, bb in ((W1a, b1a, W1b, b1b), (W2a, b2a, W2b, b2b),
                           (W3a, b3a, W3b, b3b)):
        parts = _sc_agg(h, src, dst, zeros)
        a0 = lax.slice(parts, (0, 0), (N, DF))
        a1 = lax.slice(parts, (NPAD, 0), (NPAD + N, DF))
        h = _tc_mlp(h, a0, a1, Wa, ba, Wb, bb)

    return _tc_pool_post(h, batch3, Wp1, bp1, Wp2, bp2)


# speedup vs baseline: 1.4376x; 1.0003x over previous
import functools

import jax
import jax.numpy as jnp
from jax import lax
from jax.experimental import pallas as pl
from jax.experimental.pallas import tpu as pltpu
from jax.experimental.pallas import tpu_sc as plsc

N = 10000
E = 320000
DIN = 128
DH = 64
DF = 128
DOUT = 6
G = 128

NC = 2
NS = 16
NW = NC * NS
CHUNK = 128
S0 = 152
S1 = 8
SEGS0 = (48, 48, 48, 8)
SEGS1 = (8,)
SEGMAX = 48
TCH = NS * (S0 + S1)
CH0 = NS * S0
EP = TCH * CHUNK
NPAD = 10112
RPW = NPAD // NS


def _sc_agg(h, src, dst, zeros):
    mesh = plsc.VectorSubcoreMesh(core_axis_name="c", subcore_axis_name="s")

    @functools.partial(
        pl.kernel,
        mesh=mesh,
        out_type=jax.ShapeDtypeStruct((NC * NPAD, DF), jnp.float32),
        scratch_types=[
            pltpu.VMEM((SEGMAX, CHUNK), jnp.int32),
            pltpu.VMEM((SEGMAX, CHUNK), jnp.int32),
            pltpu.VMEM((CHUNK, DF), jnp.float32),
            pltpu.VMEM((CHUNK, DF), jnp.float32),
            pltpu.VMEM_SHARED((NPAD, DF), jnp.float32),
            pltpu.SemaphoreType.DMA,
            pltpu.SemaphoreType.DMA,
            pltpu.SemaphoreType.DMA,
            pltpu.SemaphoreType.DMA,
            pltpu.SemaphoreType.DMA,
        ],
    )
    def k(h_hbm, src_hbm, dst_hbm, z_hbm, out_hbm, srcv, dstv,
          rows0, rows1, acc, isem, gs0, gs1, ss0, ss1):
        c = lax.axis_index("c")
        s = lax.axis_index("s")
        pltpu.sync_copy(z_hbm.at[pl.ds(s * RPW, RPW)], acc.at[pl.ds(s * RPW, RPW)])
        plsc.subcore_barrier()

        def run(base_chunk, segs):
            off = 0
            for hsteps in segs:
                b = base_chunk + off
                off += hsteps
                ic0 = pltpu.async_copy(
                    src_hbm.at[pl.ds(b, hsteps)], srcv.at[pl.ds(0, hsteps)],
                    isem)
                ic1 = pltpu.async_copy(
                    dst_hbm.at[pl.ds(b, hsteps)], dstv.at[pl.ds(0, hsteps)],
                    isem)
                ic0.wait()
                ic1.wait()

                @pl.loop(0, hsteps, step=2)
                def _(t):
                    g0 = pltpu.async_copy(h_hbm.at[srcv.at[t]], rows0, gs0)
                    g1 = pltpu.async_copy(h_hbm.at[srcv.at[t + 1]], rows1, gs1)
                    g0.wait()
                    s0 = pltpu.async_copy(rows0, acc.at[dstv.at[t]], ss0,
                                          add=True)
                    g1.wait()
                    s1 = pltpu.async_copy(rows1, acc.at[dstv.at[t + 1]], ss1,
                                          add=True)
                    s0.wait()
                    s1.wait()

        @pl.when(c == 0)
        def _():
            run(s * S0, SEGS0)

        @pl.when(c == 1)
        def _():
            run(CH0 + s * S1, SEGS1)

        plsc.subcore_barrier()
        pltpu.sync_copy(
            acc.at[pl.ds(s * RPW, RPW)],
            out_hbm.at[pl.ds(c * NPAD + s * RPW, RPW)],
        )

    return k(h, src, dst, zeros)


_ROWS = 2000


def _tc_pre(x, W, b):
    def body(x_ref, w_ref, b_ref, o_ref):
        t = (
            jnp.dot(x_ref[...], w_ref[...], preferred_element_type=jnp.float32)
            + b_ref[...]
        )
        o_ref[...] = jnp.concatenate(
            [t, jnp.zeros((_ROWS, DF - DH), jnp.float32)], axis=1
        )

    return pl.pallas_call(
        body,
        grid=(N // _ROWS,),
        in_specs=[
            pl.BlockSpec((_ROWS, DIN), lambda i: (i, 0)),
            pl.BlockSpec((DIN, DH), lambda i: (0, 0)),
            pl.BlockSpec((1, DH), lambda i: (0, 0)),
        ],
        out_specs=pl.BlockSpec((_ROWS, DF), lambda i: (i, 0)),
        out_shape=jax.ShapeDtypeStruct((N, DF), jnp.float32),
    )(x, W, b.reshape(1, DH))


def _tc_mlp(h, a0, a1, Wa, ba, Wb, bb):
    def body(h_ref, a0_ref, a1_ref, wa, bar, wb, bbr, o_ref):
        t = h_ref[...] + a0_ref[...] + a1_ref[...]
        t = t[:, :DH]
        t = jnp.maximum(
            jnp.dot(t, wa[...], preferred_element_type=jnp.float32) + bar[...], 0.0
        )
        t = jnp.dot(t, wb[...], preferred_element_type=jnp.float32) + bbr[...]
        t = jnp.maximum(t, 0.0)
        o_ref[...] = jnp.concatenate(
            [t, jnp.zeros((_ROWS, DF - DH), jnp.float32)], axis=1
        )

    rows_spec = pl.BlockSpec((_ROWS, DF), lambda i: (i, 0))
    w_spec = pl.BlockSpec((DH, DH), lambda i: (0, 0))
    b_spec = pl.BlockSpec((1, DH), lambda i: (0, 0))
    return pl.pallas_call(
        body,
        grid=(N // _ROWS,),
        in_specs=[rows_spec, rows_spec, rows_spec, w_spec, b_spec, w_spec, b_spec],
        out_specs=rows_spec,
        out_shape=jax.ShapeDtypeStruct((N, DF), jnp.float32),
    )(h, a0, a1, Wa, ba.reshape(1, DH), Wb, bb.reshape(1, DH))


def _tc_pool_post(h, batch3, Wp1, bp1, Wp2, bp2):
    nb = N // _ROWS

    def body(h_ref, b_ref, w1, b1r, w2, b2r, o_ref, acc):
        i = pl.program_id(0)

        @pl.when(i == 0)
        def _():
            acc[...] = jnp.zeros_like(acc)

        ids = b_ref[0]
        gi = lax.broadcasted_iota(jnp.int32, (G, _ROWS), 0)
        onehot = (gi == ids).astype(jnp.float32)
        acc[...] += jnp.dot(
            onehot, h_ref[...][:, :DH], preferred_element_type=jnp.float32
        )

        @pl.when(i == nb - 1)
        def _():
            p = acc[...]
            t = jnp.maximum(
                jnp.dot(p, w1[...], preferred_element_type=jnp.float32) + b1r[...],
                0.0,
            )
            o = jnp.dot(t, w2[...], preferred_element_type=jnp.float32) + b2r[...]
            m = jnp.max(o, axis=1, keepdims=True)
            lse = jnp.log(jnp.sum(jnp.exp(o - m), axis=1, keepdims=True)) + m
            o_ref[...] = o - lse

    return pl.pallas_call(
        body,
        grid=(nb,),
        in_specs=[
            pl.BlockSpec((_ROWS, DF), lambda i: (i, 0)),
            pl.BlockSpec((1, 1, _ROWS), lambda i: (i, 0, 0)),
            pl.BlockSpec((DH, DH), lambda i: (0, 0)),
            pl.BlockSpec((1, DH), lambda i: (0, 0)),
            pl.BlockSpec((DH, DOUT), lambda i: (0, 0)),
            pl.BlockSpec((1, DOUT), lambda i: (0, 0)),
        ],
        out_specs=pl.BlockSpec((G, DOUT), lambda i: (0, 0)),
        out_shape=jax.ShapeDtypeStruct((G, DOUT), jnp.float32),
        scratch_shapes=[pltpu.VMEM((G, DH), jnp.float32)],
    )(h, batch3, Wp1, bp1.reshape(1, DH), Wp2, bp2.reshape(1, DOUT))


def kernel(x, edge_index, batch, W_pre, b_pre, W1a, b1a, W1b, b1b, W2a, b2a,
           W2b, b2b, W3a, b3a, W3b, b3b, Wp1, bp1, Wp2, bp2):
    pad = EP - E
    src = jnp.concatenate([edge_index[0], jnp.zeros((pad,), jnp.int32)])
    pad_dst = N + jnp.arange(pad, dtype=jnp.int32) % (NPAD - N)
    dst = jnp.concatenate([edge_index[1], pad_dst])
    src = src.reshape(TCH, CHUNK)
    dst = dst.reshape(TCH, CHUNK)
    zeros = jnp.zeros((NPAD, DF), jnp.float32)
    batch3 = batch.reshape(N // _ROWS, 1, _ROWS)

    h = _tc_pre(x, W_pre, b_pre)
    for Wa, ba, Wb, bb in ((W1a, b1a, W1b, b1b), (W2a, b2a, W2b, b2b),
                           (W3a, b3a, W3b, b3b)):
        parts = _sc_agg(h, src, dst, zeros)
        a0 = lax.slice(parts, (0, 0), (N, DF))
        a1 = lax.slice(parts, (NPAD, 0), (NPAD + N, DF))
        h = _tc_mlp(h, a0, a1, Wa, ba, Wb, bb)

    return _tc_pool_post(h, batch3, Wp1, bp1, Wp2, bp2)
